# baseline (speedup 1.0000x reference)
import functools

import jax
import jax.numpy as jnp
from jax import lax
from jax.experimental import pallas as pl
from jax.experimental.pallas import tpu as pltpu
from jax.experimental.pallas import tpu_sc as plsc

NUM_EXPERTS = 8
T = 2048
D_IN = 768
D_HID = 1024
D_OUT = 768

BLK = 256
NB = T // BLK
NT = NB + NUM_EXPERTS - 1


def _routing_metadata(indices_s):
    idx = indices_s.astype(jnp.int32)
    oh = (idx[:, None] == jnp.arange(NUM_EXPERTS, dtype=jnp.int32)[None, :]
          ).astype(jnp.int32)
    ch = oh.reshape(T // 128, 128, NUM_EXPERTS)
    r = jax.lax.broadcasted_iota(jnp.int32, (128, 128), 0)
    c = jax.lax.broadcasted_iota(jnp.int32, (128, 128), 1)
    tri = (r >= c).astype(jnp.float32)
    within_ch = jnp.einsum("rc,kce->kre", tri, ch.astype(jnp.float32),
                           preferred_element_type=jnp.float32)
    chunk_tot = jnp.sum(ch, axis=1)
    chunk_pre = jnp.concatenate(
        [jnp.zeros((1, NUM_EXPERTS), jnp.int32),
         jnp.cumsum(chunk_tot[:-1], axis=0).astype(jnp.int32)])
    within = (within_ch.astype(jnp.int32)
              + chunk_pre[:, None, :]).reshape(T, NUM_EXPERTS)
    counts = jnp.sum(chunk_tot, axis=0)
    offsets = jnp.concatenate(
        [jnp.zeros((1,), jnp.int32), jnp.cumsum(counts).astype(jnp.int32)])
    inv = jnp.sum(oh * (within - 1 + offsets[:-1][None, :]), axis=1)

    cuts = jnp.sort(jnp.concatenate(
        [jnp.arange(NB, dtype=jnp.int32) * BLK, offsets[1:NUM_EXPERTS]]))
    next_cuts = jnp.concatenate([cuts[1:], jnp.array([T], jnp.int32)])
    block_id = jnp.minimum(cuts // BLK, NB - 1)
    expert_id = jnp.sum(
        (cuts[:, None] >= offsets[None, 1:NUM_EXPERTS]).astype(jnp.int32),
        axis=1)
    row_start = cuts - block_id * BLK
    row_end = next_cuts - block_id * BLK
    first = jnp.concatenate(
        [jnp.ones((1,), jnp.int32),
         (block_id[1:] != block_id[:-1]).astype(jnp.int32)])
    return inv, block_id, expert_id, row_start, row_end, first


@functools.lru_cache(maxsize=None)
def _make_sc_permute(n_rows, n_cols, scatter):
    info = plsc.get_sparse_core_info()
    nw = info.num_cores * info.num_subcores
    b_per_w = n_rows // nw
    mesh = plsc.VectorSubcoreMesh(core_axis_name="c", subcore_axis_name="s")

    @functools.partial(
        pl.kernel, mesh=mesh,
        out_type=jax.ShapeDtypeStruct((n_rows, n_cols), jnp.float32),
        scratch_types=[
            pltpu.VMEM((b_per_w,), jnp.int32),
            pltpu.VMEM((b_per_w, n_cols), jnp.float32),
            pltpu.SemaphoreType.DMA,
        ],
        compiler_params=pltpu.CompilerParams(use_tc_tiling_on_sc=True),
    )
    def permute(table_hbm, idx_hbm, out_hbm, idx_v, rows_v, sem):
        wid = lax.axis_index("s") * info.num_cores + lax.axis_index("c")
        base = wid * b_per_w
        pltpu.sync_copy(idx_hbm.at[pl.ds(base, b_per_w)], idx_v)
        if scatter:
            pltpu.sync_copy(table_hbm.at[pl.ds(base, b_per_w)], rows_v)
            pltpu.async_copy(rows_v, out_hbm.at[idx_v], sem).wait()
        else:
            pltpu.async_copy(table_hbm.at[idx_v], rows_v, sem).wait()
            pltpu.sync_copy(rows_v, out_hbm.at[pl.ds(base, b_per_w)])

    return permute


def _ffn_tile_body(bid_ref, eid_ref, rs_ref, re_ref, fr_ref,
                   xs_ref, w1_ref, w2_ref, b1_ref, b2_ref, out_ref):
    i = pl.program_id(0)
    x = xs_ref[...].astype(jnp.bfloat16)
    h = jnp.dot(x, w1_ref[0].astype(jnp.bfloat16),
                preferred_element_type=jnp.float32)
    h = h + b1_ref[0]
    h = 0.5 * h * (1.0 + lax.erf(h * 0.7071067811865476))
    o = jnp.dot(h.astype(jnp.bfloat16), w2_ref[0].astype(jnp.bfloat16),
                preferred_element_type=jnp.float32)
    o = o + b2_ref[0]
    rows = lax.broadcasted_iota(jnp.int32, (BLK, 1), 0)
    mask = (rows >= rs_ref[i]) & (rows < re_ref[i])
    contrib = jnp.where(mask, o, 0.0)

    @pl.when(fr_ref[i] == 1)
    def _():
        out_ref[...] = contrib

    @pl.when(fr_ref[i] == 0)
    def _():
        out_ref[...] = out_ref[...] + contrib


def _grouped_ffn(xs, weight1, weight2, bias1, bias2,
                 block_id, expert_id, row_start, row_end, first):
    grid_spec = pltpu.PrefetchScalarGridSpec(
        num_scalar_prefetch=5,
        grid=(NT,),
        in_specs=[
            pl.BlockSpec((BLK, D_IN),
                         lambda i, bid, eid, rs, re, fr: (bid[i], 0)),
            pl.BlockSpec((1, D_IN, D_HID),
                         lambda i, bid, eid, rs, re, fr: (eid[i], 0, 0)),
            pl.BlockSpec((1, D_HID, D_OUT),
                         lambda i, bid, eid, rs, re, fr: (eid[i], 0, 0)),
            pl.BlockSpec((1, 1, D_HID),
                         lambda i, bid, eid, rs, re, fr: (eid[i], 0, 0)),
            pl.BlockSpec((1, 1, D_OUT),
                         lambda i, bid, eid, rs, re, fr: (eid[i], 0, 0)),
        ],
        out_specs=pl.BlockSpec((BLK, D_OUT),
                               lambda i, bid, eid, rs, re, fr: (bid[i], 0)),
    )
    return pl.pallas_call(
        _ffn_tile_body,
        grid_spec=grid_spec,
        out_shape=jax.ShapeDtypeStruct((T, D_OUT), jnp.float32),
        compiler_params=pltpu.CompilerParams(
            dimension_semantics=("arbitrary",)),
    )(block_id, expert_id, row_start, row_end, first,
      xs, weight1, weight2,
      bias1.reshape(NUM_EXPERTS, 1, D_HID),
      bias2.reshape(NUM_EXPERTS, 1, D_OUT))


def kernel(x, indices_s, weight1, weight2, bias1, bias2):
    warm = _make_sc_permute(256, 128, False)(
        jnp.zeros((256, 128), jnp.float32), jnp.arange(256, dtype=jnp.int32))
    warm_zero = jnp.minimum(jnp.abs(warm[0, 0]), 0.0).astype(jnp.int32)

    (inv, block_id, expert_id,
     row_start, row_end, first) = _routing_metadata(indices_s)
    xs = _make_sc_permute(T, D_IN, True)(x, inv + warm_zero)
    out_sorted = _grouped_ffn(xs, weight1, weight2, bias1, bias2,
                              block_id, expert_id, row_start, row_end, first)
    out = _make_sc_permute(T, D_OUT, False)(out_sorted, inv)
    return out[:, None, :]


# speedup vs baseline: 1.0297x; 1.0297x over previous
import functools

import jax
import jax.numpy as jnp
from jax import lax
from jax.experimental import pallas as pl
from jax.experimental.pallas import tpu as pltpu
from jax.experimental.pallas import tpu_sc as plsc

NUM_EXPERTS = 8
T = 2048
D_IN = 768
D_HID = 1024
D_OUT = 768

BLK = 128
NB = T // BLK
NT = NB + NUM_EXPERTS - 1


def _routing_metadata(indices_s):
    idx = indices_s.astype(jnp.int32)
    oh = (idx[:, None] == jnp.arange(NUM_EXPERTS, dtype=jnp.int32)[None, :]
          ).astype(jnp.int32)
    ch = oh.reshape(T // 128, 128, NUM_EXPERTS)
    r = jax.lax.broadcasted_iota(jnp.int32, (128, 128), 0)
    c = jax.lax.broadcasted_iota(jnp.int32, (128, 128), 1)
    tri = (r >= c).astype(jnp.float32)
    within_ch = jnp.einsum("rc,kce->kre", tri, ch.astype(jnp.float32),
                           preferred_element_type=jnp.float32)
    chunk_tot = jnp.sum(ch, axis=1)
    chunk_pre = jnp.concatenate(
        [jnp.zeros((1, NUM_EXPERTS), jnp.int32),
         jnp.cumsum(chunk_tot[:-1], axis=0).astype(jnp.int32)])
    within = (within_ch.astype(jnp.int32)
              + chunk_pre[:, None, :]).reshape(T, NUM_EXPERTS)
    counts = jnp.sum(chunk_tot, axis=0)
    offsets = jnp.concatenate(
        [jnp.zeros((1,), jnp.int32), jnp.cumsum(counts).astype(jnp.int32)])
    inv = jnp.sum(oh * (within - 1 + offsets[:-1][None, :]), axis=1)

    cuts = jnp.sort(jnp.concatenate(
        [jnp.arange(NB, dtype=jnp.int32) * BLK, offsets[1:NUM_EXPERTS]]))
    next_cuts = jnp.concatenate([cuts[1:], jnp.array([T], jnp.int32)])
    block_id = jnp.minimum(cuts // BLK, NB - 1)
    expert_id = jnp.sum(
        (cuts[:, None] >= offsets[None, 1:NUM_EXPERTS]).astype(jnp.int32),
        axis=1)
    row_start = cuts - block_id * BLK
    row_end = next_cuts - block_id * BLK
    first = jnp.concatenate(
        [jnp.ones((1,), jnp.int32),
         (block_id[1:] != block_id[:-1]).astype(jnp.int32)])
    return inv, block_id, expert_id, row_start, row_end, first


@functools.lru_cache(maxsize=None)
def _make_sc_permute(n_rows, n_cols, scatter):
    info = plsc.get_sparse_core_info()
    nw = info.num_cores * info.num_subcores
    b_per_w = n_rows // nw
    mesh = plsc.VectorSubcoreMesh(core_axis_name="c", subcore_axis_name="s")

    @functools.partial(
        pl.kernel, mesh=mesh,
        out_type=jax.ShapeDtypeStruct((n_rows, n_cols), jnp.float32),
        scratch_types=[
            pltpu.VMEM((b_per_w,), jnp.int32),
            pltpu.VMEM((b_per_w, n_cols), jnp.float32),
            pltpu.SemaphoreType.DMA,
        ],
        compiler_params=pltpu.CompilerParams(use_tc_tiling_on_sc=True),
    )
    def permute(table_hbm, idx_hbm, out_hbm, idx_v, rows_v, sem):
        wid = lax.axis_index("s") * info.num_cores + lax.axis_index("c")
        base = wid * b_per_w
        pltpu.sync_copy(idx_hbm.at[pl.ds(base, b_per_w)], idx_v)
        if scatter:
            pltpu.sync_copy(table_hbm.at[pl.ds(base, b_per_w)], rows_v)
            pltpu.async_copy(rows_v, out_hbm.at[idx_v], sem).wait()
        else:
            pltpu.async_copy(table_hbm.at[idx_v], rows_v, sem).wait()
            pltpu.sync_copy(rows_v, out_hbm.at[pl.ds(base, b_per_w)])

    return permute


def _ffn_tile_body(bid_ref, eid_ref, rs_ref, re_ref, fr_ref,
                   xs_ref, w1_ref, w2_ref, b1_ref, b2_ref, out_ref):
    i = pl.program_id(0)
    x = xs_ref[...].astype(jnp.bfloat16)
    h = jnp.dot(x, w1_ref[0].astype(jnp.bfloat16),
                preferred_element_type=jnp.float32)
    h = h + b1_ref[0]
    h = 0.5 * h * (1.0 + lax.erf(h * 0.7071067811865476))
    o = jnp.dot(h.astype(jnp.bfloat16), w2_ref[0].astype(jnp.bfloat16),
                preferred_element_type=jnp.float32)
    o = o + b2_ref[0]
    rows = lax.broadcasted_iota(jnp.int32, (BLK, 1), 0)
    mask = (rows >= rs_ref[i]) & (rows < re_ref[i])
    contrib = jnp.where(mask, o, 0.0)

    @pl.when(fr_ref[i] == 1)
    def _():
        out_ref[...] = contrib

    @pl.when(fr_ref[i] == 0)
    def _():
        out_ref[...] = out_ref[...] + contrib


def _grouped_ffn(xs, weight1, weight2, bias1, bias2,
                 block_id, expert_id, row_start, row_end, first):
    grid_spec = pltpu.PrefetchScalarGridSpec(
        num_scalar_prefetch=5,
        grid=(NT,),
        in_specs=[
            pl.BlockSpec((BLK, D_IN),
                         lambda i, bid, eid, rs, re, fr: (bid[i], 0)),
            pl.BlockSpec((1, D_IN, D_HID),
                         lambda i, bid, eid, rs, re, fr: (eid[i], 0, 0)),
            pl.BlockSpec((1, D_HID, D_OUT),
                         lambda i, bid, eid, rs, re, fr: (eid[i], 0, 0)),
            pl.BlockSpec((1, 1, D_HID),
                         lambda i, bid, eid, rs, re, fr: (eid[i], 0, 0)),
            pl.BlockSpec((1, 1, D_OUT),
                         lambda i, bid, eid, rs, re, fr: (eid[i], 0, 0)),
        ],
        out_specs=pl.BlockSpec((BLK, D_OUT),
                               lambda i, bid, eid, rs, re, fr: (bid[i], 0)),
    )
    return pl.pallas_call(
        _ffn_tile_body,
        grid_spec=grid_spec,
        out_shape=jax.ShapeDtypeStruct((T, D_OUT), jnp.float32),
        compiler_params=pltpu.CompilerParams(
            dimension_semantics=("arbitrary",)),
    )(block_id, expert_id, row_start, row_end, first,
      xs, weight1, weight2,
      bias1.reshape(NUM_EXPERTS, 1, D_HID),
      bias2.reshape(NUM_EXPERTS, 1, D_OUT))


def kernel(x, indices_s, weight1, weight2, bias1, bias2):
    (inv, block_id, expert_id,
     row_start, row_end, first) = _routing_metadata(indices_s)
    xs = _make_sc_permute(T, D_IN, True)(x, inv)
    out_sorted = _grouped_ffn(xs, weight1, weight2, bias1, bias2,
                              block_id, expert_id, row_start, row_end, first)
    out = _make_sc_permute(T, D_OUT, False)(out_sorted, inv)
    return out[:, None, :]


# speedup vs baseline: 1.1303x; 1.0977x over previous
import functools

import jax
import jax.numpy as jnp
from jax import lax
from jax.experimental import pallas as pl
from jax.experimental.pallas import tpu as pltpu
from jax.experimental.pallas import tpu_sc as plsc

NUM_EXPERTS = 8
T = 2048
D_IN = 768
D_HID = 1024
D_OUT = 768

BLK = 256
NB = T // BLK
NT = NB + NUM_EXPERTS - 1
NSLOT = 4
NLOOK = 3


def _routing_metadata(indices_s):
    idx = indices_s.astype(jnp.int32)
    oh = (idx[:, None] == jnp.arange(NUM_EXPERTS, dtype=jnp.int32)[None, :]
          ).astype(jnp.int32)
    ch = oh.reshape(T // 128, 128, NUM_EXPERTS)
    r = jax.lax.broadcasted_iota(jnp.int32, (128, 128), 0)
    c = jax.lax.broadcasted_iota(jnp.int32, (128, 128), 1)
    tri = (r >= c).astype(jnp.float32)
    within_ch = jnp.einsum("rc,kce->kre", tri, ch.astype(jnp.float32),
                           preferred_element_type=jnp.float32)
    chunk_tot = jnp.sum(ch, axis=1)
    chunk_pre = jnp.concatenate(
        [jnp.zeros((1, NUM_EXPERTS), jnp.int32),
         jnp.cumsum(chunk_tot[:-1], axis=0).astype(jnp.int32)])
    within = (within_ch.astype(jnp.int32)
              + chunk_pre[:, None, :]).reshape(T, NUM_EXPERTS)
    counts = jnp.sum(chunk_tot, axis=0)
    offsets = jnp.concatenate(
        [jnp.zeros((1,), jnp.int32), jnp.cumsum(counts).astype(jnp.int32)])
    inv = jnp.sum(oh * (within - 1 + offsets[:-1][None, :]), axis=1)

    cuts = jnp.sort(jnp.concatenate(
        [jnp.arange(NB, dtype=jnp.int32) * BLK, offsets[1:NUM_EXPERTS]]))
    next_cuts = jnp.concatenate([cuts[1:], jnp.array([T], jnp.int32)])
    block_id = jnp.minimum(cuts // BLK, NB - 1)
    expert_id = jnp.sum(
        (cuts[:, None] >= offsets[None, 1:NUM_EXPERTS]).astype(jnp.int32),
        axis=1)
    row_start = cuts - block_id * BLK
    row_end = next_cuts - block_id * BLK
    first = jnp.concatenate(
        [jnp.ones((1,), jnp.int32),
         (block_id[1:] != block_id[:-1]).astype(jnp.int32)])

    run_start = jnp.concatenate(
        [jnp.ones((1,), jnp.int32),
         (expert_id[1:] != expert_id[:-1]).astype(jnp.int32)])
    run_id = jnp.cumsum(run_start).astype(jnp.int32) - 1
    num_runs = run_id[-1] + 1
    expert_of_run = jnp.zeros((NT,), jnp.int32).at[run_id].set(expert_id)
    pf_run = run_id + NLOOK
    pf_valid = (pf_run < num_runs).astype(jnp.int32)
    pf_expert = expert_of_run[jnp.minimum(pf_run, NT - 1)]
    slot = run_id % NSLOT
    pf_slot = pf_run % NSLOT
    num_runs_arr = jnp.full((1,), num_runs, jnp.int32)
    return (inv, block_id, expert_id, row_start, row_end, first,
            run_start, slot, pf_slot, pf_expert, pf_valid,
            expert_of_run, num_runs_arr)


@functools.lru_cache(maxsize=None)
def _make_sc_permute(n_rows, n_cols, scatter):
    info = plsc.get_sparse_core_info()
    nw = info.num_cores * info.num_subcores
    b_per_w = n_rows // nw
    mesh = plsc.VectorSubcoreMesh(core_axis_name="c", subcore_axis_name="s")

    @functools.partial(
        pl.kernel, mesh=mesh,
        out_type=jax.ShapeDtypeStruct((n_rows, n_cols), jnp.float32),
        scratch_types=[
            pltpu.VMEM((b_per_w,), jnp.int32),
            pltpu.VMEM((b_per_w, n_cols), jnp.float32),
            pltpu.SemaphoreType.DMA,
        ],
        compiler_params=pltpu.CompilerParams(use_tc_tiling_on_sc=True),
    )
    def permute(table_hbm, idx_hbm, out_hbm, idx_v, rows_v, sem):
        wid = lax.axis_index("s") * info.num_cores + lax.axis_index("c")
        base = wid * b_per_w
        pltpu.sync_copy(idx_hbm.at[pl.ds(base, b_per_w)], idx_v)
        if scatter:
            pltpu.sync_copy(table_hbm.at[pl.ds(base, b_per_w)], rows_v)
            pltpu.async_copy(rows_v, out_hbm.at[idx_v], sem).wait()
        else:
            pltpu.async_copy(table_hbm.at[idx_v], rows_v, sem).wait()
            pltpu.sync_copy(rows_v, out_hbm.at[pl.ds(base, b_per_w)])

    return permute


def _ffn_tile_body(bid_ref, eid_ref, rs_ref, re_ref, fr_ref,
                   rst_ref, slot_ref, pfs_ref, pfe_ref, pfv_ref,
                   eor_ref, nrun_ref,
                   xs_ref, w1_any, w2_any, b1_ref, b2_ref, out_ref,
                   w1buf, w2buf, sem1, sem2):
    i = pl.program_id(0)
    s = slot_ref[i]

    def _issue(e, sl):
        pltpu.make_async_copy(w1_any.at[e], w1buf.at[sl], sem1.at[sl]).start()
        pltpu.make_async_copy(w2_any.at[e], w2buf.at[sl], sem2.at[sl]).start()

    @pl.when(i == 0)
    def _prologue():
        _issue(eor_ref[0], 0)
        for r in range(1, NLOOK):
            @pl.when(nrun_ref[0] > r)
            def _():
                _issue(eor_ref[r], r % NSLOT)

    @pl.when(rst_ref[i] == 1)
    def _run_start():
        @pl.when(pfv_ref[i] == 1)
        def _():
            _issue(pfe_ref[i], pfs_ref[i])
        pltpu.make_async_copy(w1_any.at[0], w1buf.at[s], sem1.at[s]).wait()
        pltpu.make_async_copy(w2_any.at[0], w2buf.at[s], sem2.at[s]).wait()

    x = xs_ref[...].astype(jnp.bfloat16)
    h = jnp.dot(x, w1buf[s].astype(jnp.bfloat16),
                preferred_element_type=jnp.float32)
    h = h + b1_ref[0]
    h = 0.5 * h * (1.0 + lax.erf(h * 0.7071067811865476))
    o = jnp.dot(h.astype(jnp.bfloat16), w2buf[s].astype(jnp.bfloat16),
                preferred_element_type=jnp.float32)
    o = o + b2_ref[0]
    rows = lax.broadcasted_iota(jnp.int32, (BLK, 1), 0)
    mask = (rows >= rs_ref[i]) & (rows < re_ref[i])
    contrib = jnp.where(mask, o, 0.0)

    @pl.when(fr_ref[i] == 1)
    def _():
        out_ref[...] = contrib

    @pl.when(fr_ref[i] == 0)
    def _():
        out_ref[...] = out_ref[...] + contrib


def _grouped_ffn(xs, weight1, weight2, bias1, bias2, meta):
    (block_id, expert_id, row_start, row_end, first,
     run_start, slot, pf_slot, pf_expert, pf_valid,
     expert_of_run, num_runs_arr) = meta
    grid_spec = pltpu.PrefetchScalarGridSpec(
        num_scalar_prefetch=12,
        grid=(NT,),
        in_specs=[
            pl.BlockSpec((BLK, D_IN),
                         lambda i, bid, *_: (bid[i], 0)),
            pl.BlockSpec(memory_space=pltpu.MemorySpace.HBM),
            pl.BlockSpec(memory_space=pltpu.MemorySpace.HBM),
            pl.BlockSpec((1, 1, D_HID),
                         lambda i, bid, eid, *_: (eid[i], 0, 0)),
            pl.BlockSpec((1, 1, D_OUT),
                         lambda i, bid, eid, *_: (eid[i], 0, 0)),
        ],
        out_specs=pl.BlockSpec((BLK, D_OUT),
                               lambda i, bid, *_: (bid[i], 0)),
        scratch_shapes=[
            pltpu.VMEM((NSLOT, D_IN, D_HID), jnp.float32),
            pltpu.VMEM((NSLOT, D_HID, D_OUT), jnp.float32),
            pltpu.SemaphoreType.DMA((NSLOT,)),
            pltpu.SemaphoreType.DMA((NSLOT,)),
        ],
    )
    return pl.pallas_call(
        _ffn_tile_body,
        grid_spec=grid_spec,
        out_shape=jax.ShapeDtypeStruct((T, D_OUT), jnp.float32),
        compiler_params=pltpu.CompilerParams(
            dimension_semantics=("arbitrary",)),
    )(block_id, expert_id, row_start, row_end, first,
      run_start, slot, pf_slot, pf_expert, pf_valid,
      expert_of_run, num_runs_arr,
      xs, weight1, weight2,
      bias1.reshape(NUM_EXPERTS, 1, D_HID),
      bias2.reshape(NUM_EXPERTS, 1, D_OUT))


def kernel(x, indices_s, weight1, weight2, bias1, bias2):
    inv, *meta = _routing_metadata(indices_s)
    xs = _make_sc_permute(T, D_IN, True)(x, inv)
    out_sorted = _grouped_ffn(xs, weight1, weight2, bias1, bias2, meta)
    out = _make_sc_permute(T, D_OUT, False)(out_sorted, inv)
    return out[:, None, :]
